# baseline (speedup 1.0000x reference)
import functools

import jax
import jax.numpy as jnp
from jax import lax
from jax.experimental import pallas as pl
from jax.experimental.pallas import tpu as pltpu

N = 10000
E = 320000
H = 128
NR = 6
OE = 256
OC = 1
L = 4
NG = 128
CUTOFF = 5.0
PEXP = 5

EBLK = 3200
NBLK = 2000


def _silu(v):
    return v * jax.nn.sigmoid(v)


def _prep_kernel(z_ref, emb_ref, we1_ref, we2_ref, x_ref, a_ref, b_ref):
    z = z_ref[...]
    oh = (z == lax.broadcasted_iota(jnp.int32, (NBLK, 128), 1)).astype(jnp.float32)
    x = jnp.dot(oh, emb_ref[...], preferred_element_type=jnp.float32)
    x_ref[...] = x
    a_ref[...] = jnp.dot(x, we1_ref[...], preferred_element_type=jnp.float32)
    b_ref[...] = jnp.dot(x, we2_ref[...], preferred_element_type=jnp.float32)


def _prep_nodes(z2d, emb_pad, we1, we2):
    grid = N // NBLK
    return pl.pallas_call(
        _prep_kernel,
        grid=(grid,),
        in_specs=[
            pl.BlockSpec((NBLK, 1), lambda i: (i, 0)),
            pl.BlockSpec((128, H), lambda i: (0, 0)),
            pl.BlockSpec((H, H), lambda i: (0, 0)),
            pl.BlockSpec((H, H), lambda i: (0, 0)),
        ],
        out_specs=[
            pl.BlockSpec((NBLK, H), lambda i: (i, 0)),
            pl.BlockSpec((NBLK, H), lambda i: (i, 0)),
            pl.BlockSpec((NBLK, H), lambda i: (i, 0)),
        ],
        out_shape=[
            jax.ShapeDtypeStruct((N, H), jnp.float32),
            jax.ShapeDtypeStruct((N, H), jnp.float32),
            jax.ShapeDtypeStruct((N, H), jnp.float32),
        ],
    )(z2d, emb_pad, we1, we2)


def _edge_init_kernel(ps_ref, pd_ref, asrc_ref, bdst_ref, wrbf_ref, wfold_ref,
                      be_ref, w10_ref, rbfh_ref, e_ref, m_ref):
    dvec = ps_ref[...] - pd_ref[...]
    d2 = jnp.sum(dvec * dvec, axis=1)
    d = jnp.sqrt(d2 + 1e-9)
    u = d / CUTOFF
    p = PEXP
    a_c = -(p + 1) * (p + 2) / 2.0
    b_c = p * (p + 2)
    c_c = -p * (p + 1) / 2.0
    env = 1.0 + a_c * u ** p + b_c * u ** (p + 1) + c_c * u ** (p + 2)
    env = jnp.where(u < 1.0, env, 0.0)
    nv = lax.broadcasted_iota(jnp.int32, (EBLK, 8), 1).astype(jnp.float32) + 1.0
    rbf = (env / (d + 1e-9))[:, None] * jnp.sin(nv * (jnp.pi * u[:, None]))
    rbf_h = jnp.dot(rbf, wrbf_ref[...], preferred_element_type=jnp.float32)
    rbfh_ref[...] = rbf_h
    pre = (asrc_ref[...] + bdst_ref[...] + be_ref[...]
           + jnp.dot(rbf, wfold_ref[...], preferred_element_type=jnp.float32))
    e = _silu(pre)
    e_ref[...] = e
    m_ref[...] = _silu(jnp.dot(e, w10_ref[...],
                               preferred_element_type=jnp.float32)) * rbf_h


def _edge_init(ps, pd, asrc, bdst, wrbf_pad, wfold_pad, be2d, w10):
    grid = E // EBLK
    eb = lambda i: (i, 0)
    w = lambda i: (0, 0)
    return pl.pallas_call(
        _edge_init_kernel,
        grid=(grid,),
        in_specs=[
            pl.BlockSpec((EBLK, 8), eb),
            pl.BlockSpec((EBLK, 8), eb),
            pl.BlockSpec((EBLK, H), eb),
            pl.BlockSpec((EBLK, H), eb),
            pl.BlockSpec((8, H), w),
            pl.BlockSpec((8, H), w),
            pl.BlockSpec((1, H), w),
            pl.BlockSpec((H, H), w),
        ],
        out_specs=[
            pl.BlockSpec((EBLK, H), eb),
            pl.BlockSpec((EBLK, H), eb),
            pl.BlockSpec((EBLK, H), eb),
        ],
        out_shape=[
            jax.ShapeDtypeStruct((E, H), jnp.float32),
            jax.ShapeDtypeStruct((E, H), jnp.float32),
            jax.ShapeDtypeStruct((E, H), jnp.float32),
        ],
    )(ps, pd, asrc, bdst, wrbf_pad, wfold_pad, be2d, w10)


def _node_kernel(x_ref, agg_ref, w2_ref, w3_ref, xo_ref, c_ref):
    xn = x_ref[...] + _silu(jnp.dot(agg_ref[...], w2_ref[...],
                                    preferred_element_type=jnp.float32))
    xo_ref[...] = xn
    c_ref[...] = jnp.dot(xn, w3_ref[...], preferred_element_type=jnp.float32)


def _node_update(x, agg, w2l, w3l):
    grid = N // NBLK
    nb = lambda i: (i, 0)
    w = lambda i: (0, 0)
    return pl.pallas_call(
        _node_kernel,
        grid=(grid,),
        in_specs=[
            pl.BlockSpec((NBLK, H), nb),
            pl.BlockSpec((NBLK, H), nb),
            pl.BlockSpec((H, H), w),
            pl.BlockSpec((H, H), w),
        ],
        out_specs=[
            pl.BlockSpec((NBLK, H), nb),
            pl.BlockSpec((NBLK, H), nb),
        ],
        out_shape=[
            jax.ShapeDtypeStruct((N, H), jnp.float32),
            jax.ShapeDtypeStruct((N, H), jnp.float32),
        ],
    )(x, agg, w2l, w3l)


def _edge_kernel(e_ref, cs_ref, cd_ref, rbfh_ref, w1_ref, eo_ref, m_ref):
    en = _silu(e_ref[...] + cs_ref[...] + cd_ref[...])
    eo_ref[...] = en
    m_ref[...] = _silu(jnp.dot(en, w1_ref[...],
                               preferred_element_type=jnp.float32)) * rbfh_ref[...]


def _edge_update(e, cs, cd, rbf_h, w1n):
    grid = E // EBLK
    eb = lambda i: (i, 0)
    w = lambda i: (0, 0)
    return pl.pallas_call(
        _edge_kernel,
        grid=(grid,),
        in_specs=[
            pl.BlockSpec((EBLK, H), eb),
            pl.BlockSpec((EBLK, H), eb),
            pl.BlockSpec((EBLK, H), eb),
            pl.BlockSpec((EBLK, H), eb),
            pl.BlockSpec((H, H), w),
        ],
        out_specs=[
            pl.BlockSpec((EBLK, H), eb),
            pl.BlockSpec((EBLK, H), eb),
        ],
        out_shape=[
            jax.ShapeDtypeStruct((E, H), jnp.float32),
            jax.ShapeDtypeStruct((E, H), jnp.float32),
        ],
    )(e, cs, cd, rbf_h, w1n)


def _readout_kernel(x_ref, b_ref, wo1_ref, wo2_ref, out_ref):
    h = _silu(jnp.dot(x_ref[...], wo1_ref[...],
                      preferred_element_type=jnp.float32))
    s = jnp.dot(h, wo2_ref[...], preferred_element_type=jnp.float32)
    oh = (b_ref[...] == lax.broadcasted_iota(jnp.int32, (N, NG), 1)).astype(jnp.float32)
    out_ref[...] = lax.dot_general(oh, s, (((0,), (0,)), ((), ())),
                                   preferred_element_type=jnp.float32)


def _readout(x, batch2d, wo1, wo2_pad):
    return pl.pallas_call(
        _readout_kernel,
        grid=(1,),
        in_specs=[
            pl.BlockSpec((N, H), lambda i: (0, 0)),
            pl.BlockSpec((N, 1), lambda i: (0, 0)),
            pl.BlockSpec((H, OE), lambda i: (0, 0)),
            pl.BlockSpec((OE, 8), lambda i: (0, 0)),
        ],
        out_specs=pl.BlockSpec((NG, 8), lambda i: (0, 0)),
        out_shape=jax.ShapeDtypeStruct((NG, 8), jnp.float32),
    )(x, batch2d, wo1, wo2_pad)


def kernel(z, pos, edge_index, batch, emb_table, W_rbf, W_e, b_e, W1, W2, W3,
           W_out1, W_out2):
    src = edge_index[0]
    dst = edge_index[1]

    emb_pad = jnp.zeros((128, H), jnp.float32).at[:emb_table.shape[0]].set(emb_table)
    we1 = W_e[:H]
    we2 = W_e[H:2 * H]
    wfold_pad = jnp.zeros((8, H), jnp.float32).at[:NR].set(W_rbf @ W_e[2 * H:])
    wrbf_pad = jnp.zeros((8, H), jnp.float32).at[:NR].set(W_rbf)
    pos_pad = jnp.zeros((N, 8), jnp.float32).at[:, :3].set(pos)
    wo2_pad = jnp.zeros((OE, 8), jnp.float32).at[:, :OC].set(W_out2)

    x, a, b = _prep_nodes(z.astype(jnp.int32).reshape(N, 1), emb_pad, we1, we2)

    ps = jnp.take(pos_pad, src, axis=0)
    pd = jnp.take(pos_pad, dst, axis=0)
    asrc = jnp.take(a, src, axis=0)
    bdst = jnp.take(b, dst, axis=0)

    rbf_h, e, m = _edge_init(ps, pd, asrc, bdst, wrbf_pad, wfold_pad,
                             b_e.reshape(1, H), W1[0])

    for l in range(L):
        agg = jax.ops.segment_sum(m, dst, num_segments=N)
        x, c = _node_update(x, agg, W2[l], W3[l])
        if l < L - 1:
            cs = jnp.take(c, src, axis=0)
            cd = jnp.take(c, dst, axis=0)
            e, m = _edge_update(e, cs, cd, rbf_h, W1[l + 1])

    out = _readout(x, batch.astype(jnp.int32).reshape(N, 1), W_out1, wo2_pad)
    return out[:, :OC]


# speedup vs baseline: 3.0628x; 3.0628x over previous
import functools

import jax
import jax.numpy as jnp
from jax import lax
from jax.experimental import pallas as pl
from jax.experimental.pallas import tpu as pltpu
from jax.experimental.pallas import tpu_sc as plsc

N = 10000
E = 320000
H = 128
NR = 6
OE = 256
OC = 1
L = 4
NG = 128
CUTOFF = 5.0
PEXP = 5

EBLK = 3200
NBLK = 2000


def _silu(v):
    return v * jax.nn.sigmoid(v)


NC = 2
NS = 16
NW = NC * NS
RPW = E // NW
CH = 40
NCH = RPW // CH
NPAD = 10240
NPW = NPAD // NS

_MESH = plsc.VectorSubcoreMesh(core_axis_name="c", subcore_axis_name="s",
                               num_cores=NC, num_subcores=NS)


def _wid():
    return lax.axis_index("s") * NC + lax.axis_index("c")


def _gather2_body(d, ta_ref, ia_ref, tb_ref, ib_ref, oa_ref, ob_ref,
                  ia_v, ib_v, a0, a1, b0, b1, sa0, sa1, sb0, sb1):
    wid = _wid()
    base = wid * RPW
    pltpu.sync_copy(ia_ref.at[wid], ia_v)
    pltpu.sync_copy(ib_ref.at[wid], ib_v)

    def group(g, carry):
        j0 = g * 2
        j1 = j0 + 1
        ca0 = pltpu.async_copy(ta_ref.at[ia_v.at[j0]], a0, sa0)
        cb0 = pltpu.async_copy(tb_ref.at[ib_v.at[j0]], b0, sb0)
        ca1 = pltpu.async_copy(ta_ref.at[ia_v.at[j1]], a1, sa1)
        cb1 = pltpu.async_copy(tb_ref.at[ib_v.at[j1]], b1, sb1)
        ca0.wait()
        pltpu.sync_copy(a0, oa_ref.at[pl.ds(base + j0 * CH, CH)])
        cb0.wait()
        pltpu.sync_copy(b0, ob_ref.at[pl.ds(base + j0 * CH, CH)])
        ca1.wait()
        pltpu.sync_copy(a1, oa_ref.at[pl.ds(base + j1 * CH, CH)])
        cb1.wait()
        pltpu.sync_copy(b1, ob_ref.at[pl.ds(base + j1 * CH, CH)])
        return carry

    lax.fori_loop(0, NCH // 2, group, 0)


def _sc_gather2(tab_a, idx_a, tab_b, idx_b):
    d = tab_a.shape[1]
    f = functools.partial(_gather2_body, d)
    return pl.kernel(
        f,
        out_type=[
            jax.ShapeDtypeStruct((E, d), jnp.float32),
            jax.ShapeDtypeStruct((E, d), jnp.float32),
        ],
        mesh=_MESH,
        scratch_types=[
            pltpu.VMEM((NCH, CH), jnp.int32),
            pltpu.VMEM((NCH, CH), jnp.int32),
            pltpu.VMEM((CH, d), jnp.float32),
            pltpu.VMEM((CH, d), jnp.float32),
            pltpu.VMEM((CH, d), jnp.float32),
            pltpu.VMEM((CH, d), jnp.float32),
            pltpu.SemaphoreType.DMA,
            pltpu.SemaphoreType.DMA,
            pltpu.SemaphoreType.DMA,
            pltpu.SemaphoreType.DMA,
        ],
    )(tab_a, idx_a.reshape(NW, NCH, CH), tab_b, idx_b.reshape(NW, NCH, CH))


def _scatter_body(m_ref, idx_ref, zero_ref, out_ref,
                  idx_v, r0, r1, s0, s1, acc):
    wid = _wid()
    cid = lax.axis_index("c")
    sid = lax.axis_index("s")
    base = wid * RPW
    pltpu.sync_copy(idx_ref.at[wid], idx_v)
    pltpu.sync_copy(zero_ref, acc.at[pl.ds(sid * NPW, NPW)])
    plsc.subcore_barrier()

    def group(g, carry):
        j0 = g * 2
        j1 = j0 + 1
        c0 = pltpu.async_copy(m_ref.at[pl.ds(base + j0 * CH, CH)], r0, s0)
        c1 = pltpu.async_copy(m_ref.at[pl.ds(base + j1 * CH, CH)], r1, s1)
        c0.wait()
        pltpu.sync_copy(r0, acc.at[idx_v.at[j0]], add=True)
        c1.wait()
        pltpu.sync_copy(r1, acc.at[idx_v.at[j1]], add=True)
        return carry

    lax.fori_loop(0, NCH // 2, group, 0)
    plsc.subcore_barrier()
    pltpu.sync_copy(acc.at[pl.ds(sid * NPW, NPW)],
                    out_ref.at[cid, pl.ds(sid * NPW, NPW)])


def _sc_scatter_add(m, dst, zeros):
    return pl.kernel(
        _scatter_body,
        out_type=jax.ShapeDtypeStruct((NC, NPAD, H), jnp.float32),
        mesh=_MESH,
        scratch_types=[
            pltpu.VMEM((NCH, CH), jnp.int32),
            pltpu.VMEM((CH, H), jnp.float32),
            pltpu.VMEM((CH, H), jnp.float32),
            pltpu.SemaphoreType.DMA,
            pltpu.SemaphoreType.DMA,
            pltpu.VMEM_SHARED((NPAD, H), jnp.float32),
        ],
    )(m, dst.reshape(NW, NCH, CH), zeros)


def _prep_kernel(z_ref, pos_ref, emb_ref, we1_ref, we2_ref,
                 x_ref, ta_ref, tb_ref):
    z = z_ref[...]
    oh = (z == lax.broadcasted_iota(jnp.int32, (NBLK, 128), 1)).astype(jnp.float32)
    x = jnp.dot(oh, emb_ref[...], preferred_element_type=jnp.float32)
    x_ref[...] = x
    a = jnp.dot(x, we1_ref[...], preferred_element_type=jnp.float32)
    b = jnp.dot(x, we2_ref[...], preferred_element_type=jnp.float32)
    pp = pos_ref[...]
    n2 = jnp.sum(pp * pp, axis=1, keepdims=True)
    px = pp[:, 0:1]
    py = pp[:, 1:2]
    pz = pp[:, 2:3]
    li = lax.broadcasted_iota(jnp.int32, (NBLK, H), 1)
    one = jnp.float32(1.0)
    p_feat = (jnp.where(li == 0, n2, 0.0) + jnp.where(li == 1, one, 0.0)
              + jnp.where(li == 2, px, 0.0) + jnp.where(li == 3, py, 0.0)
              + jnp.where(li == 4, pz, 0.0))
    q_feat = (jnp.where(li == 0, one, 0.0) + jnp.where(li == 1, n2, 0.0)
              + jnp.where(li == 2, -2.0 * px, 0.0)
              + jnp.where(li == 3, -2.0 * py, 0.0)
              + jnp.where(li == 4, -2.0 * pz, 0.0))
    ta_ref[...] = jnp.concatenate([a, p_feat], axis=1)
    tb_ref[...] = jnp.concatenate([b, q_feat], axis=1)


def _prep_nodes(z2d, pos_pad, emb_pad, we1, we2):
    grid = N // NBLK
    return pl.pallas_call(
        _prep_kernel,
        grid=(grid,),
        in_specs=[
            pl.BlockSpec((NBLK, 1), lambda i: (i, 0)),
            pl.BlockSpec((NBLK, 8), lambda i: (i, 0)),
            pl.BlockSpec((128, H), lambda i: (0, 0)),
            pl.BlockSpec((H, H), lambda i: (0, 0)),
            pl.BlockSpec((H, H), lambda i: (0, 0)),
        ],
        out_specs=[
            pl.BlockSpec((NBLK, H), lambda i: (i, 0)),
            pl.BlockSpec((NBLK, 2 * H), lambda i: (i, 0)),
            pl.BlockSpec((NBLK, 2 * H), lambda i: (i, 0)),
        ],
        out_shape=[
            jax.ShapeDtypeStruct((N, H), jnp.float32),
            jax.ShapeDtypeStruct((N, 2 * H), jnp.float32),
            jax.ShapeDtypeStruct((N, 2 * H), jnp.float32),
        ],
    )(z2d, pos_pad, emb_pad, we1, we2)


def _edge_init_kernel(ga_ref, gb_ref, wrbf_ref, wfold_ref,
                      be_ref, w10_ref, rbfh_ref, e_ref, m_ref):
    ga = ga_ref[...]
    gb = gb_ref[...]
    d2 = jnp.sum(ga[:, H:H + 8] * gb[:, H:H + 8], axis=1)
    d = jnp.sqrt(d2 + 1e-9)
    u = d / CUTOFF
    p = PEXP
    a_c = -(p + 1) * (p + 2) / 2.0
    b_c = p * (p + 2)
    c_c = -p * (p + 1) / 2.0
    env = 1.0 + a_c * u ** p + b_c * u ** (p + 1) + c_c * u ** (p + 2)
    env = jnp.where(u < 1.0, env, 0.0)
    nv = lax.broadcasted_iota(jnp.int32, (EBLK, 8), 1).astype(jnp.float32) + 1.0
    rbf = (env / (d + 1e-9))[:, None] * jnp.sin(nv * (jnp.pi * u[:, None]))
    rbf_h = jnp.dot(rbf, wrbf_ref[...], preferred_element_type=jnp.float32)
    rbfh_ref[...] = rbf_h
    pre = (ga[:, :H] + gb[:, :H] + be_ref[...]
           + jnp.dot(rbf, wfold_ref[...], preferred_element_type=jnp.float32))
    e = _silu(pre)
    e_ref[...] = e
    m_ref[...] = _silu(jnp.dot(e, w10_ref[...],
                               preferred_element_type=jnp.float32)) * rbf_h


def _edge_init(ga, gb, wrbf_pad, wfold_pad, be2d, w10):
    grid = E // EBLK
    eb = lambda i: (i, 0)
    w = lambda i: (0, 0)
    return pl.pallas_call(
        _edge_init_kernel,
        grid=(grid,),
        in_specs=[
            pl.BlockSpec((EBLK, 2 * H), eb),
            pl.BlockSpec((EBLK, 2 * H), eb),
            pl.BlockSpec((8, H), w),
            pl.BlockSpec((8, H), w),
            pl.BlockSpec((1, H), w),
            pl.BlockSpec((H, H), w),
        ],
        out_specs=[
            pl.BlockSpec((EBLK, H), eb),
            pl.BlockSpec((EBLK, H), eb),
            pl.BlockSpec((EBLK, H), eb),
        ],
        out_shape=[
            jax.ShapeDtypeStruct((E, H), jnp.float32),
            jax.ShapeDtypeStruct((E, H), jnp.float32),
            jax.ShapeDtypeStruct((E, H), jnp.float32),
        ],
    )(ga, gb, wrbf_pad, wfold_pad, be2d, w10)


def _node_kernel(x_ref, agg_ref, w2_ref, w3_ref, xo_ref, c_ref):
    agg = agg_ref[0] + agg_ref[1]
    xn = x_ref[...] + _silu(jnp.dot(agg, w2_ref[...],
                                    preferred_element_type=jnp.float32))
    xo_ref[...] = xn
    c_ref[...] = jnp.dot(xn, w3_ref[...], preferred_element_type=jnp.float32)


def _node_update(x, agg, w2l, w3l):
    grid = N // NBLK
    nb = lambda i: (i, 0)
    w = lambda i: (0, 0)
    return pl.pallas_call(
        _node_kernel,
        grid=(grid,),
        in_specs=[
            pl.BlockSpec((NBLK, H), nb),
            pl.BlockSpec((NC, NBLK, H), lambda i: (0, i, 0)),
            pl.BlockSpec((H, H), w),
            pl.BlockSpec((H, H), w),
        ],
        out_specs=[
            pl.BlockSpec((NBLK, H), nb),
            pl.BlockSpec((NBLK, H), nb),
        ],
        out_shape=[
            jax.ShapeDtypeStruct((N, H), jnp.float32),
            jax.ShapeDtypeStruct((N, H), jnp.float32),
        ],
    )(x, agg, w2l, w3l)


def _edge_kernel(e_ref, cs_ref, cd_ref, rbfh_ref, w1_ref, eo_ref, m_ref):
    en = _silu(e_ref[...] + cs_ref[...] + cd_ref[...])
    eo_ref[...] = en
    m_ref[...] = _silu(jnp.dot(en, w1_ref[...],
                               preferred_element_type=jnp.float32)) * rbfh_ref[...]


def _edge_update(e, cs, cd, rbf_h, w1n):
    grid = E // EBLK
    eb = lambda i: (i, 0)
    w = lambda i: (0, 0)
    return pl.pallas_call(
        _edge_kernel,
        grid=(grid,),
        in_specs=[
            pl.BlockSpec((EBLK, H), eb),
            pl.BlockSpec((EBLK, H), eb),
            pl.BlockSpec((EBLK, H), eb),
            pl.BlockSpec((EBLK, H), eb),
            pl.BlockSpec((H, H), w),
        ],
        out_specs=[
            pl.BlockSpec((EBLK, H), eb),
            pl.BlockSpec((EBLK, H), eb),
        ],
        out_shape=[
            jax.ShapeDtypeStruct((E, H), jnp.float32),
            jax.ShapeDtypeStruct((E, H), jnp.float32),
        ],
    )(e, cs, cd, rbf_h, w1n)


def _readout_kernel(x_ref, b_ref, wo1_ref, wo2_ref, out_ref):
    h = _silu(jnp.dot(x_ref[...], wo1_ref[...],
                      preferred_element_type=jnp.float32))
    s = jnp.dot(h, wo2_ref[...], preferred_element_type=jnp.float32)
    oh = (b_ref[...] == lax.broadcasted_iota(jnp.int32, (N, NG), 1)).astype(jnp.float32)
    out_ref[...] = lax.dot_general(oh, s, (((0,), (0,)), ((), ())),
                                   preferred_element_type=jnp.float32)


def _readout(x, batch2d, wo1, wo2_pad):
    return pl.pallas_call(
        _readout_kernel,
        grid=(1,),
        in_specs=[
            pl.BlockSpec((N, H), lambda i: (0, 0)),
            pl.BlockSpec((N, 1), lambda i: (0, 0)),
            pl.BlockSpec((H, OE), lambda i: (0, 0)),
            pl.BlockSpec((OE, 8), lambda i: (0, 0)),
        ],
        out_specs=pl.BlockSpec((NG, 8), lambda i: (0, 0)),
        out_shape=jax.ShapeDtypeStruct((NG, 8), jnp.float32),
    )(x, batch2d, wo1, wo2_pad)


def kernel(z, pos, edge_index, batch, emb_table, W_rbf, W_e, b_e, W1, W2, W3,
           W_out1, W_out2):
    src = edge_index[0]
    dst = edge_index[1]

    emb_pad = jnp.zeros((128, H), jnp.float32).at[:emb_table.shape[0]].set(emb_table)
    we1 = W_e[:H]
    we2 = W_e[H:2 * H]
    wfold_pad = jnp.zeros((8, H), jnp.float32).at[:NR].set(W_rbf @ W_e[2 * H:])
    wrbf_pad = jnp.zeros((8, H), jnp.float32).at[:NR].set(W_rbf)
    pos_pad = jnp.zeros((N, 8), jnp.float32).at[:, :3].set(pos)
    wo2_pad = jnp.zeros((OE, 8), jnp.float32).at[:, :OC].set(W_out2)

    x, ta, tb = _prep_nodes(z.astype(jnp.int32).reshape(N, 1), pos_pad,
                            emb_pad, we1, we2)

    src = src.astype(jnp.int32)
    dst = dst.astype(jnp.int32)
    zeros = jnp.zeros((NPW, H), jnp.float32)

    ga, gb = _sc_gather2(ta, src, tb, dst)

    rbf_h, e, m = _edge_init(ga, gb, wrbf_pad, wfold_pad,
                             b_e.reshape(1, H), W1[0])

    for l in range(L):
        agg = _sc_scatter_add(m, dst, zeros)
        x, c = _node_update(x, agg, W2[l], W3[l])
        if l < L - 1:
            cs, cd = _sc_gather2(c, src, c, dst)
            e, m = _edge_update(e, cs, cd, rbf_h, W1[l + 1])

    out = _readout(x, batch.astype(jnp.int32).reshape(N, 1), W_out1, wo2_pad)
    return out[:, :OC]
